# baseline (speedup 1.0000x reference)
import jax
import jax.numpy as jnp
from jax.experimental import pallas as pl
from jax.experimental.pallas import tpu as pltpu


def _body(z_ref, out_ref):
    v = z_ref[0, 0, 0]
    out_ref[...] = jnp.full((16, 128, 3, 3), v, jnp.float32)


def kernel(z, w2, b2, w1, b1):
    zr = z.reshape(64, 64, 64)
    out = pl.pallas_call(
        _body,
        grid=(64, 8),
        in_specs=[pl.BlockSpec((1, 64, 64), lambda h, q: (h, 0, 0))],
        out_specs=pl.BlockSpec((16, 128, 3, 3), lambda h, q: (h, q, 0, 0)),
        out_shape=jax.ShapeDtypeStruct((1024, 1024, 3, 3), jnp.float32),
        compiler_params=pltpu.CompilerParams(
            dimension_semantics=("parallel", "arbitrary"),
        ),
    )(zr)
    return out


# speedup vs baseline: 9.3531x; 9.3531x over previous
import numpy as np
import jax
import jax.numpy as jnp
from jax.experimental import pallas as pl
from jax.experimental.pallas import tpu as pltpu

H, K = 64, 64
Z = 64
OUT = 16
C = 144
G = 8
P = 9
HB = 2
SROWS = 2 * P * G


def _body(z_ref, w2_ref, b2_ref, sel_ref, wsh_ref, b1sh_ref, out_ref):
    zb = jnp.concatenate([z_ref[0], z_ref[1]], axis=0)
    a = jnp.dot(zb, w2_ref[...], preferred_element_type=jnp.float32)
    a = a + b2_ref[...]
    sel = jnp.dot(sel_ref[...], a, preferred_element_type=jnp.float32)
    b1sh = b1sh_ref[...]
    for p in range(P):
        wp = wsh_ref[:, p * 128:(p + 1) * 128]
        bp = b1sh[:, p * 128:(p + 1) * 128]
        pieces = []
        for hh in range(HB):
            s1 = sel[hh * SROWS + p * G:hh * SROWS + (p + 1) * G]
            s2 = sel[hh * SROWS + (P + p) * G:hh * SROWS + (P + p + 1) * G]
            for o in range(OUT):
                pieces.append(jnp.concatenate(
                    [s1[:, o * Z:(o + 1) * Z], s2[:, o * Z:(o + 1) * Z]],
                    axis=1))
        lhs = jnp.concatenate(pieces, axis=0)
        t = jnp.dot(lhs, wp, preferred_element_type=jnp.float32)
        for r in range(HB * OUT):
            out_ref[r, :, p * 128:(p + 1) * 128] = t[r * G:(r + 1) * G] + bp


def _shifted_weights(w1, b1):
    j = np.arange(P * 128)
    c = j % C
    klocal = j // C
    kl = 128 * (j // 128) // C
    top = np.where(klocal == kl, 1.0, 0.0).astype(np.float32)
    bot = np.where(klocal == kl + 1, 1.0, 0.0).astype(np.float32)
    w1c = w1[:, c]
    wsh = jnp.concatenate([w1c * top[None, :], w1c * bot[None, :]], axis=0)
    b1sh = b1[c][None, :]
    return wsh, b1sh


def _selection_matrix():
    sel = np.zeros((HB * SROWS, HB * K), np.float32)
    for hh in range(HB):
        for p in range(P):
            kl = (128 * p) // C
            for g in range(G):
                sel[hh * SROWS + p * G + g, hh * K + G * g + kl] = 1.0
                sel[hh * SROWS + (P + p) * G + g,
                    hh * K + (G * g + kl + 1) % K] = 1.0
    return jnp.asarray(sel)


def kernel(z, w2, b2, w1, b1):
    zr = z.reshape(H, K, Z)
    b2r = b2.reshape(1, OUT * Z)
    wsh, b1sh = _shifted_weights(w1, b1)
    selm = _selection_matrix()
    out = pl.pallas_call(
        _body,
        grid=(H // HB,),
        in_specs=[
            pl.BlockSpec((HB, K, Z), lambda i: (i, 0, 0)),
            pl.BlockSpec((Z, OUT * Z), lambda i: (0, 0)),
            pl.BlockSpec((1, OUT * Z), lambda i: (0, 0)),
            pl.BlockSpec((HB * SROWS, HB * K), lambda i: (0, 0)),
            pl.BlockSpec((2 * Z, P * 128), lambda i: (0, 0)),
            pl.BlockSpec((1, P * 128), lambda i: (0, 0)),
        ],
        out_specs=pl.BlockSpec((HB * OUT, G, P * 128), lambda i: (i, 0, 0)),
        out_shape=jax.ShapeDtypeStruct((H * OUT, G, P * 128), jnp.float32),
        compiler_params=pltpu.CompilerParams(
            dimension_semantics=("parallel",),
        ),
    )(zr, w2, b2r, selm, wsh, b1sh)
    return out.reshape(H * OUT, K * 16, 3, 3)


# speedup vs baseline: 9.8869x; 1.0571x over previous
import numpy as np
import jax
import jax.numpy as jnp
from jax.experimental import pallas as pl
from jax.experimental.pallas import tpu as pltpu

H, K = 64, 64
Z = 64
OUT = 16
C = 144
G = 8
P = 9
HB = 4
SROWS = 2 * P * G


def _body(z_ref, w2_ref, b2_ref, sel_ref, wsh_ref, b1sh_ref, out_ref):
    zb = jnp.concatenate([z_ref[i] for i in range(HB)], axis=0)
    a = jnp.dot(zb, w2_ref[...], preferred_element_type=jnp.float32)
    a = a + b2_ref[...]
    sel = jnp.dot(sel_ref[...], a, preferred_element_type=jnp.float32)
    b1sh = b1sh_ref[...]
    for p in range(P):
        wp = wsh_ref[:, p * 128:(p + 1) * 128]
        bp = b1sh[:, p * 128:(p + 1) * 128]
        pieces = []
        for hh in range(HB):
            s1 = sel[hh * SROWS + p * G:hh * SROWS + (p + 1) * G]
            s2 = sel[hh * SROWS + (P + p) * G:hh * SROWS + (P + p + 1) * G]
            for o in range(OUT):
                pieces.append(jnp.concatenate(
                    [s1[:, o * Z:(o + 1) * Z], s2[:, o * Z:(o + 1) * Z]],
                    axis=1))
        lhs = jnp.concatenate(pieces, axis=0)
        t = jnp.dot(lhs, wp, preferred_element_type=jnp.float32)
        for r in range(HB * OUT):
            out_ref[r, :, p * 128:(p + 1) * 128] = t[r * G:(r + 1) * G] + bp


def _shifted_weights(w1, b1):
    j = np.arange(P * 128)
    c = j % C
    klocal = j // C
    kl = 128 * (j // 128) // C
    top = np.where(klocal == kl, 1.0, 0.0).astype(np.float32)
    bot = np.where(klocal == kl + 1, 1.0, 0.0).astype(np.float32)
    w1c = w1[:, c]
    wsh = jnp.concatenate([w1c * top[None, :], w1c * bot[None, :]], axis=0)
    b1sh = b1[c][None, :]
    return wsh, b1sh


def _selection_matrix():
    sel = np.zeros((HB * SROWS, HB * K), np.float32)
    for hh in range(HB):
        for p in range(P):
            kl = (128 * p) // C
            for g in range(G):
                sel[hh * SROWS + p * G + g, hh * K + G * g + kl] = 1.0
                sel[hh * SROWS + (P + p) * G + g,
                    hh * K + (G * g + kl + 1) % K] = 1.0
    return jnp.asarray(sel)


def kernel(z, w2, b2, w1, b1):
    zr = z.reshape(H, K, Z)
    b2r = b2.reshape(1, OUT * Z)
    wsh, b1sh = _shifted_weights(w1, b1)
    selm = _selection_matrix()
    out = pl.pallas_call(
        _body,
        grid=(H // HB,),
        in_specs=[
            pl.BlockSpec((HB, K, Z), lambda i: (i, 0, 0)),
            pl.BlockSpec((Z, OUT * Z), lambda i: (0, 0)),
            pl.BlockSpec((1, OUT * Z), lambda i: (0, 0)),
            pl.BlockSpec((HB * SROWS, HB * K), lambda i: (0, 0)),
            pl.BlockSpec((2 * Z, P * 128), lambda i: (0, 0)),
            pl.BlockSpec((1, P * 128), lambda i: (0, 0)),
        ],
        out_specs=pl.BlockSpec((HB * OUT, G, P * 128), lambda i: (i, 0, 0)),
        out_shape=jax.ShapeDtypeStruct((H * OUT, G, P * 128), jnp.float32),
        compiler_params=pltpu.CompilerParams(
            dimension_semantics=("parallel",),
        ),
    )(zr, w2, b2r, selm, wsh, b1sh)
    return out.reshape(H * OUT, K * 16, 3, 3)
